# baseline (speedup 1.0000x reference)
import functools

import jax
import jax.numpy as jnp
from jax import lax
from jax.experimental import pallas as pl
from jax.experimental.pallas import tpu as pltpu
from jax.experimental.pallas import tpu_sc as plsc

_B = 16384
_NW = 32
_BPW = _B // _NW
_CHUNK = 128
_NCH = _BPW // _CHUNK
_TD = 32
_AD = 32


def _sc_gather_body(idx_t_hbm, idx_a_hbm, ttab_hbm, atab_hbm,
                    t_out, a_out, idx_t, idx_a, rows_t, rows_a, sem):
    wid = lax.axis_index("s") * 2 + lax.axis_index("c")
    base = wid * _BPW
    pltpu.sync_copy(idx_t_hbm.at[wid], idx_t)
    pltpu.sync_copy(idx_a_hbm.at[wid], idx_a)
    copies = []
    for j in range(_NCH):
        copies.append(pltpu.async_copy(
            ttab_hbm.at[idx_t.at[j]], rows_t.at[pl.ds(j * _CHUNK, _CHUNK)], sem))
        copies.append(pltpu.async_copy(
            atab_hbm.at[idx_a.at[j]], rows_a.at[pl.ds(j * _CHUNK, _CHUNK)], sem))
    for c in copies:
        c.wait()
    pltpu.sync_copy(rows_t, t_out.at[pl.ds(base, _BPW)])
    pltpu.sync_copy(rows_a, a_out.at[pl.ds(base, _BPW)])


def _sc_gather(task_idx3, action_idx3, task_table, action_table):
    mesh = plsc.VectorSubcoreMesh(core_axis_name="c", subcore_axis_name="s")
    f = pl.kernel(
        _sc_gather_body,
        mesh=mesh,
        out_type=(
            jax.ShapeDtypeStruct((_B, _TD), jnp.float32),
            jax.ShapeDtypeStruct((_B, _AD), jnp.float32),
        ),
        scratch_types=[
            pltpu.VMEM((_NCH, _CHUNK), jnp.int32),
            pltpu.VMEM((_NCH, _CHUNK), jnp.int32),
            pltpu.VMEM((_BPW, _TD), jnp.float32),
            pltpu.VMEM((_BPW, _AD), jnp.float32),
            pltpu.SemaphoreType.DMA,
        ],
        compiler_params=pltpu.CompilerParams(use_tc_tiling_on_sc=False),
    )
    return f(task_idx3, action_idx3, task_table, action_table)


def _tc_fuse_body(t_ref, a_ref, s_ref, o_ref, ws_ref, bs_ref, wo_ref, bo_ref,
                  out_ref):
    t = t_ref[...]
    nt = jnp.sqrt(jnp.sum(t * t, axis=1, keepdims=True))
    t = t * jnp.where(nt > 1.0, 1.0 / (nt + 1e-7), 1.0)
    a = a_ref[...]
    na = jnp.sqrt(jnp.sum(a * a, axis=1, keepdims=True))
    a = a * jnp.where(na > 1.0, 1.0 / (na + 1e-7), 1.0)
    s = jnp.dot(s_ref[...], ws_ref[...], preferred_element_type=jnp.float32)
    s = s + bs_ref[...]
    s = s / jnp.maximum(jnp.sqrt(jnp.sum(s * s, axis=1, keepdims=True)), 1e-12)
    o = jnp.dot(o_ref[...], wo_ref[...], preferred_element_type=jnp.float32)
    o = o + bo_ref[...]
    o = o / jnp.maximum(jnp.sqrt(jnp.sum(o * o, axis=1, keepdims=True)), 1e-12)
    out_ref[...] = jnp.concatenate([t, a, s, o], axis=1)


def _tc_fuse(t_rows, a_rows, state, orientation, W_state, b_state, W_orient,
             b_orient, block_rows=2048):
    nblk = _B // block_rows
    row_blk = lambda i: (i, 0)
    rep = lambda i: (0, 0)
    return pl.pallas_call(
        _tc_fuse_body,
        grid=(nblk,),
        in_specs=[
            pl.BlockSpec((block_rows, _TD), row_blk),
            pl.BlockSpec((block_rows, _AD), row_blk),
            pl.BlockSpec((block_rows, 128), row_blk),
            pl.BlockSpec((block_rows, 16), row_blk),
            pl.BlockSpec((128, 64), rep),
            pl.BlockSpec((1, 64), rep),
            pl.BlockSpec((16, 64), rep),
            pl.BlockSpec((1, 64), rep),
        ],
        out_specs=pl.BlockSpec((block_rows, _TD + _AD + 128), row_blk),
        out_shape=jax.ShapeDtypeStruct((_B, _TD + _AD + 128), jnp.float32),
    )(t_rows, a_rows, state, orientation, W_state, b_state, W_orient, b_orient)


def kernel(task, action, state, orientation, task_table, action_table,
           W_state, b_state, W_orient, b_orient):
    task_idx3 = task.reshape(_NW, _NCH, _CHUNK)
    action_idx3 = action.reshape(_NW, _NCH, _CHUNK)
    t_rows, a_rows = _sc_gather(task_idx3, action_idx3, task_table, action_table)
    return _tc_fuse(t_rows, a_rows, state, orientation,
                    W_state, b_state.reshape(1, -1), W_orient,
                    b_orient.reshape(1, -1))


# speedup vs baseline: 1.0408x; 1.0408x over previous
import functools

import jax
import jax.numpy as jnp
from jax import lax
from jax.experimental import pallas as pl
from jax.experimental.pallas import tpu as pltpu
from jax.experimental.pallas import tpu_sc as plsc

_B = 16384
_NW = 32
_BPW = _B // _NW
_CHUNK = 128
_NCH = _BPW // _CHUNK
_TD = 32
_AD = 32


def _sc_gather_body(idx_t_hbm, idx_a_hbm, ttab_hbm, atab_hbm,
                    t_out, a_out, idx_t, idx_a, rows_t, rows_a, sem):
    wid = lax.axis_index("s") * 2 + lax.axis_index("c")
    base = wid * _BPW
    pltpu.sync_copy(idx_t_hbm.at[wid], idx_t)
    pltpu.sync_copy(idx_a_hbm.at[wid], idx_a)
    copies = []
    for j in range(_NCH):
        copies.append(pltpu.async_copy(
            ttab_hbm.at[idx_t.at[j]], rows_t.at[pl.ds(j * _CHUNK, _CHUNK)], sem))
        copies.append(pltpu.async_copy(
            atab_hbm.at[idx_a.at[j]], rows_a.at[pl.ds(j * _CHUNK, _CHUNK)], sem))
    for c in copies:
        c.wait()
    pltpu.sync_copy(rows_t, t_out.at[pl.ds(base, _BPW)])
    pltpu.sync_copy(rows_a, a_out.at[pl.ds(base, _BPW)])


def _sc_gather(task_idx3, action_idx3, task_table, action_table):
    mesh = plsc.VectorSubcoreMesh(core_axis_name="c", subcore_axis_name="s")
    f = pl.kernel(
        _sc_gather_body,
        mesh=mesh,
        out_type=(
            jax.ShapeDtypeStruct((_B, _TD), jnp.float32),
            jax.ShapeDtypeStruct((_B, _AD), jnp.float32),
        ),
        scratch_types=[
            pltpu.VMEM((_NCH, _CHUNK), jnp.int32),
            pltpu.VMEM((_NCH, _CHUNK), jnp.int32),
            pltpu.VMEM((_BPW, _TD), jnp.float32),
            pltpu.VMEM((_BPW, _AD), jnp.float32),
            pltpu.SemaphoreType.DMA,
        ],
        compiler_params=pltpu.CompilerParams(use_tc_tiling_on_sc=False),
    )
    return f(task_idx3, action_idx3, task_table, action_table)


def _tc_fuse_body(t_ref, a_ref, s_ref, o_ref, ws_ref, bs_ref, wo_ref, bo_ref,
                  out_ref):
    t = t_ref[...]
    nt = jnp.sqrt(jnp.sum(t * t, axis=0, keepdims=True))
    t = t * jnp.where(nt > 1.0, 1.0 / (nt + 1e-7), 1.0)
    a = a_ref[...]
    na = jnp.sqrt(jnp.sum(a * a, axis=0, keepdims=True))
    a = a * jnp.where(na > 1.0, 1.0 / (na + 1e-7), 1.0)
    s = jax.lax.dot_general(ws_ref[...], s_ref[...], (((1,), (1,)), ((), ())),
                            preferred_element_type=jnp.float32)
    s = s + bs_ref[...]
    s = s / jnp.maximum(jnp.sqrt(jnp.sum(s * s, axis=0, keepdims=True)), 1e-12)
    o = jnp.dot(wo_ref[...], o_ref[...], preferred_element_type=jnp.float32)
    o = o + bo_ref[...]
    o = o / jnp.maximum(jnp.sqrt(jnp.sum(o * o, axis=0, keepdims=True)), 1e-12)
    out_ref[...] = jnp.concatenate([t, a, s, o], axis=0)


def _tc_fuse(t_rows_t, a_rows_t, state_t, orientation_t, W_state_t, b_state,
             W_orient_t, b_orient, block_cols=2048):
    nblk = _B // block_cols
    col_blk = lambda i: (0, i)
    rep = lambda i: (0, 0)
    return pl.pallas_call(
        _tc_fuse_body,
        grid=(nblk,),
        in_specs=[
            pl.BlockSpec((_TD, block_cols), col_blk),
            pl.BlockSpec((_AD, block_cols), col_blk),
            pl.BlockSpec((block_cols, 128), lambda i: (i, 0)),
            pl.BlockSpec((16, block_cols), col_blk),
            pl.BlockSpec((64, 128), rep),
            pl.BlockSpec((64, 1), rep),
            pl.BlockSpec((64, 16), rep),
            pl.BlockSpec((64, 1), rep),
        ],
        out_specs=pl.BlockSpec((_TD + _AD + 128, block_cols), col_blk),
        out_shape=jax.ShapeDtypeStruct((_TD + _AD + 128, _B), jnp.float32),
    )(t_rows_t, a_rows_t, state_t, orientation_t, W_state_t, b_state,
      W_orient_t, b_orient)


def kernel(task, action, state, orientation, task_table, action_table,
           W_state, b_state, W_orient, b_orient):
    task_idx3 = task.reshape(_NW, _NCH, _CHUNK)
    action_idx3 = action.reshape(_NW, _NCH, _CHUNK)
    t_rows, a_rows = _sc_gather(task_idx3, action_idx3, task_table, action_table)
    out_t = _tc_fuse(t_rows.T, a_rows.T, state, orientation.T,
                     W_state.T, b_state.reshape(-1, 1), W_orient.T,
                     b_orient.reshape(-1, 1))
    return out_t.T


# speedup vs baseline: 1.0542x; 1.0128x over previous
import functools

import jax
import jax.numpy as jnp
from jax import lax
from jax.experimental import pallas as pl
from jax.experimental.pallas import tpu as pltpu
from jax.experimental.pallas import tpu_sc as plsc

_B = 16384
_NW = 32
_BPW = _B // _NW
_CHUNK = 128
_NCH = _BPW // _CHUNK
_TD = 32
_AD = 32


def _sc_gather_body(idx_t_hbm, idx_a_hbm, ttab_hbm, atab_hbm,
                    t_out, a_out, idx_t, idx_a, rows_t, rows_a, sem):
    wid = lax.axis_index("s") * 2 + lax.axis_index("c")
    base = wid * _BPW
    pltpu.sync_copy(idx_t_hbm.at[pl.ds(base, _BPW)], idx_t)
    pltpu.sync_copy(idx_a_hbm.at[pl.ds(base, _BPW)], idx_a)
    copies = []
    for j in range(_NCH):
        copies.append(pltpu.async_copy(
            ttab_hbm.at[idx_t.at[pl.ds(j * _CHUNK, _CHUNK)]],
            rows_t.at[pl.ds(j * _CHUNK, _CHUNK)], sem))
        copies.append(pltpu.async_copy(
            atab_hbm.at[idx_a.at[pl.ds(j * _CHUNK, _CHUNK)]],
            rows_a.at[pl.ds(j * _CHUNK, _CHUNK)], sem))
    for c in copies:
        c.wait()
    pltpu.sync_copy(rows_t, t_out.at[pl.ds(base, _BPW)])
    pltpu.sync_copy(rows_a, a_out.at[pl.ds(base, _BPW)])


def _sc_gather(task_idx, action_idx, task_table, action_table):
    mesh = plsc.VectorSubcoreMesh(core_axis_name="c", subcore_axis_name="s")
    f = pl.kernel(
        _sc_gather_body,
        mesh=mesh,
        out_type=(
            jax.ShapeDtypeStruct((_B, _TD), jnp.float32),
            jax.ShapeDtypeStruct((_B, _AD), jnp.float32),
        ),
        scratch_types=[
            pltpu.VMEM((_BPW,), jnp.int32),
            pltpu.VMEM((_BPW,), jnp.int32),
            pltpu.VMEM((_BPW, _TD), jnp.float32),
            pltpu.VMEM((_BPW, _AD), jnp.float32),
            pltpu.SemaphoreType.DMA,
        ],
        compiler_params=pltpu.CompilerParams(use_tc_tiling_on_sc=False),
    )
    return f(task_idx, action_idx, task_table, action_table)


def _tc_fuse_body(t_ref, a_ref, s_ref, o_ref, ws_ref, bs_ref, wo_ref, bo_ref,
                  out_ref):
    t = t_ref[...].T
    nt = jnp.sqrt(jnp.sum(t * t, axis=0, keepdims=True))
    t = t * jnp.where(nt > 1.0, 1.0 / (nt + 1e-7), 1.0)
    a = a_ref[...].T
    na = jnp.sqrt(jnp.sum(a * a, axis=0, keepdims=True))
    a = a * jnp.where(na > 1.0, 1.0 / (na + 1e-7), 1.0)
    s = jax.lax.dot_general(ws_ref[...], s_ref[...], (((1,), (1,)), ((), ())),
                            preferred_element_type=jnp.float32)
    s = s + bs_ref[...]
    s = s / jnp.maximum(jnp.sqrt(jnp.sum(s * s, axis=0, keepdims=True)), 1e-12)
    o = jnp.dot(wo_ref[...], o_ref[...], preferred_element_type=jnp.float32)
    o = o + bo_ref[...]
    o = o / jnp.maximum(jnp.sqrt(jnp.sum(o * o, axis=0, keepdims=True)), 1e-12)
    out_ref[...] = jnp.concatenate([t, a, s, o], axis=0)


def _tc_fuse(t_rows_t, a_rows_t, state_t, orientation_t, W_state_t, b_state,
             W_orient_t, b_orient, block_cols=2048):
    nblk = _B // block_cols
    col_blk = lambda i: (0, i)
    rep = lambda i: (0, 0)
    return pl.pallas_call(
        _tc_fuse_body,
        grid=(nblk,),
        in_specs=[
            pl.BlockSpec((block_cols, _TD), lambda i: (i, 0)),
            pl.BlockSpec((block_cols, _AD), lambda i: (i, 0)),
            pl.BlockSpec((block_cols, 128), lambda i: (i, 0)),
            pl.BlockSpec((16, block_cols), col_blk),
            pl.BlockSpec((64, 128), rep),
            pl.BlockSpec((64, 1), rep),
            pl.BlockSpec((64, 16), rep),
            pl.BlockSpec((64, 1), rep),
        ],
        out_specs=pl.BlockSpec((_TD + _AD + 128, block_cols), col_blk),
        out_shape=jax.ShapeDtypeStruct((_TD + _AD + 128, _B), jnp.float32),
    )(t_rows_t, a_rows_t, state_t, orientation_t, W_state_t, b_state,
      W_orient_t, b_orient)


def kernel(task, action, state, orientation, task_table, action_table,
           W_state, b_state, W_orient, b_orient):
    t_rows, a_rows = _sc_gather(task, action, task_table, action_table)
    out_t = _tc_fuse(t_rows, a_rows, state, orientation.T,
                     W_state.T, b_state.reshape(-1, 1), W_orient.T,
                     b_orient.reshape(-1, 1))
    return out_t.T
